# baseline (speedup 1.0000x reference)
import functools

import jax
import jax.numpy as jnp
from jax import lax
from jax.experimental import pallas as pl
from jax.experimental.pallas import tpu as pltpu
from jax.experimental.pallas import tpu_sc as plsc

_N, _E, _D, _R = 10000, 320000, 128, 8
_NC, _NS = 2, 16
_NW = _NC * _NS
_K = 128
_CPT = 80
_EP = _NW * _CPT * _K
_BN = 5000
_BE = _E
_NB = _N // _BN
_NPAD = 10112
_RT = _NPAD // _NS


def _prep_body(ei_ref, et_ref, gi_ref, ds_ref):
    gi_ref[...] = et_ref[...] * _N + ei_ref[0]
    ds_ref[...] = ei_ref[1]


def _prep(ei, et):
    return pl.pallas_call(
        _prep_body,
        grid=(_E // _BE,),
        in_specs=[
            pl.BlockSpec((2, _BE), lambda i: (0, i)),
            pl.BlockSpec((_BE,), lambda i: (i,)),
        ],
        out_specs=[
            pl.BlockSpec((_BE,), lambda i: (i,)),
            pl.BlockSpec((_BE,), lambda i: (i,)),
        ],
        out_shape=[jax.ShapeDtypeStruct((_E,), jnp.int32)] * 2,
    )(ei, et)


def _mm_body(h_ref, w_ref, o_ref):
    o_ref[...] = jnp.dot(h_ref[...].astype(jnp.bfloat16), w_ref[0],
                         preferred_element_type=jnp.float32)


_TBL_SHAPES = jax.ShapeDtypeStruct(((_R + 1) * _N, _D), jnp.float32)
_TBL_OUT_SPECS = pl.BlockSpec((_BN, _D), lambda i, r: (r * _NB + i, 0))


def _table(h, wcat):
    return pl.pallas_call(
        _mm_body,
        grid=(_NB, _R + 1),
        in_specs=[
            pl.BlockSpec((_BN, _D), lambda i, r: (i, 0)),
            pl.BlockSpec((1, _D, _D), lambda i, r: (r, 0, 0)),
        ],
        out_specs=_TBL_OUT_SPECS,
        out_shape=_TBL_SHAPES,
    )(h, wcat)


def _combine_mm_body(p_ref, sl_ref, b_ref, w_ref, o_ref):
    h = jnp.maximum(p_ref[0] + p_ref[1] + sl_ref[...] + b_ref[...], 0.0)
    o_ref[...] = jnp.dot(h.astype(jnp.bfloat16), w_ref[0],
                         preferred_element_type=jnp.float32)


def _combine_table(parts, table1, brow, wcat):
    return pl.pallas_call(
        _combine_mm_body,
        grid=(_NB, _R + 1),
        in_specs=[
            pl.BlockSpec((_NC, _BN, _D), lambda i, r: (0, i, 0)),
            pl.BlockSpec((_BN, _D), lambda i, r: (_R * _NB + i, 0)),
            pl.BlockSpec((1, _D), lambda i, r: (0, 0)),
            pl.BlockSpec((1, _D, _D), lambda i, r: (r, 0, 0)),
        ],
        out_specs=_TBL_OUT_SPECS,
        out_shape=_TBL_SHAPES,
    )(parts, table1, brow, wcat)


def _final_body(p_ref, sl_ref, b_ref, o_ref):
    h = p_ref[0] + p_ref[1] + sl_ref[...] + b_ref[...]
    nrm = jnp.sqrt(jnp.sum(h * h, axis=1, keepdims=True))
    o_ref[...] = h / jnp.maximum(nrm, 1e-12)


def _final(parts, table2, brow):
    return pl.pallas_call(
        _final_body,
        grid=(_NB,),
        in_specs=[
            pl.BlockSpec((_NC, _BN, _D), lambda i: (0, i, 0)),
            pl.BlockSpec((_BN, _D), lambda i: (_R * _NB + i, 0)),
            pl.BlockSpec((1, _D), lambda i: (0, 0)),
        ],
        out_specs=pl.BlockSpec((_BN, _D), lambda i: (i, 0)),
        out_shape=jax.ShapeDtypeStruct((_N, _D), jnp.float32),
    )(parts, table2, brow)


_EC = _E // _K


def _sc_scatter(table, gidx, dst, zeros):
    mesh = plsc.VectorSubcoreMesh(core_axis_name="c", subcore_axis_name="s")

    @functools.partial(
        pl.kernel,
        mesh=mesh,
        out_type=jax.ShapeDtypeStruct((_NC, _N, _D), jnp.float32),
        scratch_types=(
            [pltpu.VMEM((_K,), jnp.int32)] * 12
            + [pltpu.VMEM((_K, _D), jnp.float32)] * 3
            + [pltpu.VMEM_SHARED((_N, _D), jnp.float32)]
            + [pltpu.SemaphoreType.DMA] * 12
        ),
    )
    def body(table_hbm, gidx_hbm, dst_hbm, zeros_hbm, out_hbm, *scr):
        idxs = scr[0:6]
        dsts = scr[6:12]
        bufs = scr[12:15]
        acc_sh = scr[15]
        semgs = scr[16:19]
        semss = scr[19:22]
        semis = scr[22:28]
        c = lax.axis_index("c")
        s = lax.axis_index("s")
        w = s * _NC + c

        @pl.when(s < 14)
        def _():
            pltpu.sync_copy(zeros_hbm.at[pl.ds(s * 624, 624)],
                            acc_sh.at[pl.ds(s * 624, 624)])

        @pl.when(s >= 14)
        def _():
            pltpu.sync_copy(zeros_hbm.at[pl.ds(8736 + (s - 14) * 632, 632)],
                            acc_sh.at[pl.ds(8736 + (s - 14) * 632, 632)])

        plsc.subcore_barrier()

        for m in range(3):
            pltpu.async_copy(gidx_hbm.at[pl.ds((w + m * _NW) * _K, _K)],
                             idxs[m], semis[m])
            pltpu.async_copy(dst_hbm.at[pl.ds((w + m * _NW) * _K, _K)],
                             dsts[m], semis[m])

        def step(t, carry):
            for m in range(6):
                s3 = m % 3
                p3 = (m - 1) % 3
                s6 = m
                p6 = (m - 1) % 6
                f6 = (m + 3) % 6
                q = 6 * t + m

                def guard(dq, q=q):
                    cond = (w + _NW * (q + dq)) < _EC
                    if m + dq < 0:
                        cond = jnp.logical_and(t > 0, cond)
                    return cond

                @pl.when(guard(-3))
                def _(s3=s3, f6=f6, q=q):
                    pltpu.make_async_copy(
                        bufs[s3], acc_sh.at[dsts[(f6)]], semss[s3]).wait()

                @pl.when(guard(3))
                def _(f6=f6, q=q):
                    nk = w + _NW * (q + 3)
                    pltpu.async_copy(gidx_hbm.at[pl.ds(nk * _K, _K)],
                                     idxs[f6], semis[f6])
                    pltpu.async_copy(dst_hbm.at[pl.ds(nk * _K, _K)],
                                     dsts[f6], semis[f6])

                @pl.when(guard(0))
                def _(s3=s3, s6=s6, q=q):
                    ck = w + _NW * q
                    pltpu.make_async_copy(gidx_hbm.at[pl.ds(ck * _K, _K)],
                                          idxs[s6], semis[s6]).wait()
                    pltpu.make_async_copy(dst_hbm.at[pl.ds(ck * _K, _K)],
                                          dsts[s6], semis[s6]).wait()
                    pltpu.async_copy(table_hbm.at[idxs[s6]], bufs[s3], semgs[s3])

                @pl.when(guard(-1))
                def _(p3=p3, p6=p6):
                    pltpu.make_async_copy(table_hbm.at[idxs[p6]],
                                          bufs[p3], semgs[p3]).wait()
                    pltpu.async_copy(bufs[p3], acc_sh.at[dsts[p6]],
                                     semss[p3], add=True)

            return carry

        lax.fori_loop(0, 14, step, 0)
        plsc.subcore_barrier()

        @pl.when(s < 14)
        def _():
            pltpu.sync_copy(acc_sh.at[pl.ds(s * 624, 624)],
                            out_hbm.at[c, pl.ds(s * 624, 624)])

        @pl.when(s >= 14)
        def _():
            pltpu.sync_copy(acc_sh.at[pl.ds(8736 + (s - 14) * 632, 632)],
                            out_hbm.at[c, pl.ds(8736 + (s - 14) * 632, 632)])

    return body(table, gidx, dst, zeros)


def kernel(in_feat, edge_index, edge_types, W1, loop1, b1, W2, loop2, b2):
    gidx, dst = _prep(edge_index.astype(jnp.int32), edge_types.astype(jnp.int32))
    zeros = jnp.zeros((_N, _D), jnp.float32)
    wcat1 = jnp.concatenate([W1, loop1[None]], axis=0).astype(jnp.bfloat16)
    wcat2 = jnp.concatenate([W2, loop2[None]], axis=0).astype(jnp.bfloat16)
    b1r = b1.reshape(1, _D)
    b2r = b2.reshape(1, _D)

    t1 = _table(in_feat, wcat1)
    p1 = _sc_scatter(t1, gidx, dst, zeros)
    t2 = _combine_table(p1, t1, b1r, wcat2)
    p2 = _sc_scatter(t2, gidx, dst, zeros)
    return _final(p2, t2, b2r)


# speedup vs baseline: 1.0324x; 1.0324x over previous
import functools

import jax
import jax.numpy as jnp
from jax import lax
from jax.experimental import pallas as pl
from jax.experimental.pallas import tpu as pltpu
from jax.experimental.pallas import tpu_sc as plsc

_N, _E, _D, _R = 10000, 320000, 128, 8
_NC, _NS = 2, 16
_NW = _NC * _NS
_K = 128
_CPT = 80
_EP = _NW * _CPT * _K
_BN = 5000
_BE = _E
_NB = _N // _BN
_NPAD = 10112
_RT = _NPAD // _NS


def _prep_body(ei_ref, et_ref, gi_ref, ds_ref):
    gi_ref[...] = et_ref[...] * _N + ei_ref[0]
    ds_ref[...] = ei_ref[1]


def _prep(ei, et):
    return pl.pallas_call(
        _prep_body,
        grid=(_E // _BE,),
        in_specs=[
            pl.BlockSpec((2, _BE), lambda i: (0, i)),
            pl.BlockSpec((_BE,), lambda i: (i,)),
        ],
        out_specs=[
            pl.BlockSpec((_BE,), lambda i: (i,)),
            pl.BlockSpec((_BE,), lambda i: (i,)),
        ],
        out_shape=[jax.ShapeDtypeStruct((_E,), jnp.int32)] * 2,
    )(ei, et)


def _mm_body(h_ref, w_ref, o_ref):
    o_ref[...] = jnp.dot(h_ref[...], w_ref[0], preferred_element_type=jnp.float32)


_TBL_SHAPES = jax.ShapeDtypeStruct(((_R + 1) * _N, _D), jnp.float32)
_TBL_OUT_SPECS = pl.BlockSpec((_BN, _D), lambda i, r: (r * _NB + i, 0))


def _table(h, wcat):
    return pl.pallas_call(
        _mm_body,
        grid=(_NB, _R + 1),
        in_specs=[
            pl.BlockSpec((_BN, _D), lambda i, r: (i, 0)),
            pl.BlockSpec((1, _D, _D), lambda i, r: (r, 0, 0)),
        ],
        out_specs=_TBL_OUT_SPECS,
        out_shape=_TBL_SHAPES,
    )(h, wcat)


def _combine_mm_body(p_ref, sl_ref, b_ref, w_ref, o_ref):
    h = jnp.maximum(p_ref[0] + p_ref[1] + sl_ref[...] + b_ref[...], 0.0)
    o_ref[...] = jnp.dot(h, w_ref[0], preferred_element_type=jnp.float32)


def _combine_table(parts, table1, brow, wcat):
    return pl.pallas_call(
        _combine_mm_body,
        grid=(_NB, _R + 1),
        in_specs=[
            pl.BlockSpec((_NC, _BN, _D), lambda i, r: (0, i, 0)),
            pl.BlockSpec((_BN, _D), lambda i, r: (_R * _NB + i, 0)),
            pl.BlockSpec((1, _D), lambda i, r: (0, 0)),
            pl.BlockSpec((1, _D, _D), lambda i, r: (r, 0, 0)),
        ],
        out_specs=_TBL_OUT_SPECS,
        out_shape=_TBL_SHAPES,
    )(parts, table1, brow, wcat)


def _final_body(p_ref, sl_ref, b_ref, o_ref):
    h = p_ref[0] + p_ref[1] + sl_ref[...] + b_ref[...]
    nrm = jnp.sqrt(jnp.sum(h * h, axis=1, keepdims=True))
    o_ref[...] = h / jnp.maximum(nrm, 1e-12)


def _final(parts, table2, brow):
    return pl.pallas_call(
        _final_body,
        grid=(_NB,),
        in_specs=[
            pl.BlockSpec((_NC, _BN, _D), lambda i: (0, i, 0)),
            pl.BlockSpec((_BN, _D), lambda i: (_R * _NB + i, 0)),
            pl.BlockSpec((1, _D), lambda i: (0, 0)),
        ],
        out_specs=pl.BlockSpec((_BN, _D), lambda i: (i, 0)),
        out_shape=jax.ShapeDtypeStruct((_N, _D), jnp.float32),
    )(parts, table2, brow)


_EC = _E // _K


def _sc_scatter(table, gidx, dst):
    mesh = plsc.VectorSubcoreMesh(core_axis_name="c", subcore_axis_name="s")

    @functools.partial(
        pl.kernel,
        mesh=mesh,
        out_type=jax.ShapeDtypeStruct((_NC, _N, _D), jnp.float32),
        scratch_types=(
            [pltpu.VMEM((_K,), jnp.int32)] * 12
            + [pltpu.VMEM((_K, _D), jnp.float32)] * 3
            + [pltpu.VMEM_SHARED((_N, _D), jnp.float32)]
            + [pltpu.SemaphoreType.DMA] * 12
        ),
    )
    def body(table_hbm, gidx_hbm, dst_hbm, out_hbm, *scr):
        idxs = scr[0:6]
        dsts = scr[6:12]
        bufs = scr[12:15]
        acc_sh = scr[15]
        semgs = scr[16:19]
        semss = scr[19:22]
        semis = scr[22:28]
        c = lax.axis_index("c")
        s = lax.axis_index("s")
        w = s * _NC + c

        for m in range(3):
            pltpu.async_copy(gidx_hbm.at[pl.ds((w + m * _NW) * _K, _K)],
                             idxs[m], semis[m])
            pltpu.async_copy(dst_hbm.at[pl.ds((w + m * _NW) * _K, _K)],
                             dsts[m], semis[m])

        def zrow(i, carry):
            for k8 in range(8):
                bufs[0][i, pl.ds(k8 * 16, 16)] = jnp.zeros((16,), jnp.float32)
            return carry

        lax.fori_loop(0, _K, zrow, 0)

        @pl.when(s < 14)
        def _():
            for k in range(4):
                pltpu.async_copy(bufs[0], acc_sh.at[pl.ds(s * 624 + k * _K, _K)],
                                 semss[0])
            pltpu.async_copy(bufs[0].at[pl.ds(0, 112)],
                             acc_sh.at[pl.ds(s * 624 + 4 * _K, 112)], semss[0])
            for k in range(4):
                pltpu.make_async_copy(bufs[0], acc_sh.at[pl.ds(s * 624 + k * _K, _K)],
                                      semss[0]).wait()
            pltpu.make_async_copy(bufs[0].at[pl.ds(0, 112)],
                                  acc_sh.at[pl.ds(s * 624 + 4 * _K, 112)],
                                  semss[0]).wait()

        @pl.when(s >= 14)
        def _():
            base = 8736 + (s - 14) * 632
            for k in range(4):
                pltpu.async_copy(bufs[0], acc_sh.at[pl.ds(base + k * _K, _K)],
                                 semss[0])
            pltpu.async_copy(bufs[0].at[pl.ds(0, 120)],
                             acc_sh.at[pl.ds(base + 4 * _K, 120)], semss[0])
            for k in range(4):
                pltpu.make_async_copy(bufs[0], acc_sh.at[pl.ds(base + k * _K, _K)],
                                      semss[0]).wait()
            pltpu.make_async_copy(bufs[0].at[pl.ds(0, 120)],
                                  acc_sh.at[pl.ds(base + 4 * _K, 120)],
                                  semss[0]).wait()

        plsc.subcore_barrier()


        def step(t, carry):
            for m in range(6):
                s3 = m % 3
                p3 = (m - 1) % 3
                s6 = m
                p6 = (m - 1) % 6
                f6 = (m + 3) % 6
                q = 6 * t + m

                def guard(dq, q=q):
                    cond = (w + _NW * (q + dq)) < _EC
                    if m + dq < 0:
                        cond = jnp.logical_and(t > 0, cond)
                    return cond

                @pl.when(guard(-3))
                def _(s3=s3, f6=f6, q=q):
                    pltpu.make_async_copy(
                        bufs[s3], acc_sh.at[dsts[(f6)]], semss[s3]).wait()

                @pl.when(guard(3))
                def _(f6=f6, q=q):
                    nk = w + _NW * (q + 3)
                    pltpu.async_copy(gidx_hbm.at[pl.ds(nk * _K, _K)],
                                     idxs[f6], semis[f6])
                    pltpu.async_copy(dst_hbm.at[pl.ds(nk * _K, _K)],
                                     dsts[f6], semis[f6])

                @pl.when(guard(0))
                def _(s3=s3, s6=s6, q=q):
                    ck = w + _NW * q
                    pltpu.make_async_copy(gidx_hbm.at[pl.ds(ck * _K, _K)],
                                          idxs[s6], semis[s6]).wait()
                    pltpu.make_async_copy(dst_hbm.at[pl.ds(ck * _K, _K)],
                                          dsts[s6], semis[s6]).wait()
                    pltpu.async_copy(table_hbm.at[idxs[s6]], bufs[s3], semgs[s3])

                @pl.when(guard(-1))
                def _(p3=p3, p6=p6):
                    pltpu.make_async_copy(table_hbm.at[idxs[p6]],
                                          bufs[p3], semgs[p3]).wait()
                    pltpu.async_copy(bufs[p3], acc_sh.at[dsts[p6]],
                                     semss[p3], add=True)

            return carry

        lax.fori_loop(0, 14, step, 0)
        plsc.subcore_barrier()

        @pl.when(s < 14)
        def _():
            pltpu.sync_copy(acc_sh.at[pl.ds(s * 624, 624)],
                            out_hbm.at[c, pl.ds(s * 624, 624)])

        @pl.when(s >= 14)
        def _():
            pltpu.sync_copy(acc_sh.at[pl.ds(8736 + (s - 14) * 632, 632)],
                            out_hbm.at[c, pl.ds(8736 + (s - 14) * 632, 632)])

    return body(table, gidx, dst)


def kernel(in_feat, edge_index, edge_types, W1, loop1, b1, W2, loop2, b2):
    gidx, dst = _prep(edge_index.astype(jnp.int32), edge_types.astype(jnp.int32))
    wcat1 = jnp.concatenate([W1, loop1[None]], axis=0)
    wcat2 = jnp.concatenate([W2, loop2[None]], axis=0)
    b1r = b1.reshape(1, _D)
    b2r = b2.reshape(1, _D)

    t1 = _table(in_feat, wcat1)
    p1 = _sc_scatter(t1, gidx, dst)
    t2 = _combine_table(p1, t1, b1r, wcat2)
    p2 = _sc_scatter(t2, gidx, dst)
    return _final(p2, t2, b2r)


# speedup vs baseline: 1.0328x; 1.0004x over previous
import functools

import jax
import jax.numpy as jnp
from jax import lax
from jax.experimental import pallas as pl
from jax.experimental.pallas import tpu as pltpu
from jax.experimental.pallas import tpu_sc as plsc

_N, _E, _D, _R = 10000, 320000, 128, 8
_NC, _NS = 2, 16
_NW = _NC * _NS
_K = 128
_BN = 5000
_BE = _E
_NB = _N // _BN


def _prep_body(ei_ref, et_ref, gi_ref, ds_ref):
    gi_ref[...] = et_ref[...] * _N + ei_ref[0]
    ds_ref[...] = ei_ref[1]


def _prep(ei, et):
    return pl.pallas_call(
        _prep_body,
        grid=(_E // _BE,),
        in_specs=[
            pl.BlockSpec((2, _BE), lambda i: (0, i)),
            pl.BlockSpec((_BE,), lambda i: (i,)),
        ],
        out_specs=[
            pl.BlockSpec((_BE,), lambda i: (i,)),
            pl.BlockSpec((_BE,), lambda i: (i,)),
        ],
        out_shape=[jax.ShapeDtypeStruct((_E,), jnp.int32)] * 2,
    )(ei, et)


def _mm_body(h_ref, w_ref, o_ref):
    o_ref[...] = jnp.dot(h_ref[...], w_ref[0], preferred_element_type=jnp.float32)


_TBL_SHAPES = jax.ShapeDtypeStruct(((_R + 1) * _N, _D), jnp.float32)
_TBL_OUT_SPECS = pl.BlockSpec((_BN, _D), lambda i, r: (r * _NB + i, 0))


def _table(h, wcat):
    return pl.pallas_call(
        _mm_body,
        grid=(_NB, _R + 1),
        in_specs=[
            pl.BlockSpec((_BN, _D), lambda i, r: (i, 0)),
            pl.BlockSpec((1, _D, _D), lambda i, r: (r, 0, 0)),
        ],
        out_specs=_TBL_OUT_SPECS,
        out_shape=_TBL_SHAPES,
    )(h, wcat)


def _combine_mm_body(p_ref, sl_ref, b_ref, w_ref, o_ref):
    h = jnp.maximum(p_ref[0] + p_ref[1] + sl_ref[...] + b_ref[...], 0.0)
    o_ref[...] = jnp.dot(h, w_ref[0], preferred_element_type=jnp.float32)


def _combine_table(parts, table1, brow, wcat):
    return pl.pallas_call(
        _combine_mm_body,
        grid=(_NB, _R + 1),
        in_specs=[
            pl.BlockSpec((_NC, _BN, _D), lambda i, r: (0, i, 0)),
            pl.BlockSpec((_BN, _D), lambda i, r: (_R * _NB + i, 0)),
            pl.BlockSpec((1, _D), lambda i, r: (0, 0)),
            pl.BlockSpec((1, _D, _D), lambda i, r: (r, 0, 0)),
        ],
        out_specs=_TBL_OUT_SPECS,
        out_shape=_TBL_SHAPES,
    )(parts, table1, brow, wcat)


def _final_body(p_ref, sl_ref, b_ref, o_ref):
    h = p_ref[0] + p_ref[1] + sl_ref[...] + b_ref[...]
    nrm = jnp.sqrt(jnp.sum(h * h, axis=1, keepdims=True))
    o_ref[...] = h / jnp.maximum(nrm, 1e-12)


def _final(parts, table2, brow):
    return pl.pallas_call(
        _final_body,
        grid=(_NB,),
        in_specs=[
            pl.BlockSpec((_NC, _BN, _D), lambda i: (0, i, 0)),
            pl.BlockSpec((_BN, _D), lambda i: (_R * _NB + i, 0)),
            pl.BlockSpec((1, _D), lambda i: (0, 0)),
        ],
        out_specs=pl.BlockSpec((_BN, _D), lambda i: (i, 0)),
        out_shape=jax.ShapeDtypeStruct((_N, _D), jnp.float32),
    )(parts, table2, brow)


_EC = _E // _K


def _sc_scatter(table, gidx, dst):
    mesh = plsc.VectorSubcoreMesh(core_axis_name="c", subcore_axis_name="s")

    @functools.partial(
        pl.kernel,
        mesh=mesh,
        out_type=jax.ShapeDtypeStruct((_NC, _N, _D), jnp.float32),
        scratch_types=(
            [pltpu.VMEM((_K,), jnp.int32)] * 12
            + [pltpu.VMEM((_K, _D), jnp.float32)] * 3
            + [pltpu.VMEM_SHARED((_N, _D), jnp.float32)]
            + [pltpu.SemaphoreType.DMA] * 12
        ),
    )
    def body(table_hbm, gidx_hbm, dst_hbm, out_hbm, *scr):
        idxs = scr[0:6]
        dsts = scr[6:12]
        bufs = scr[12:15]
        acc_sh = scr[15]
        semgs = scr[16:19]
        semss = scr[19:22]
        semis = scr[22:28]
        c = lax.axis_index("c")
        s = lax.axis_index("s")
        w = s * _NC + c

        for m in range(3):
            pltpu.async_copy(gidx_hbm.at[pl.ds((w + m * _NW) * _K, _K)],
                             idxs[m], semis[m])
            pltpu.async_copy(dst_hbm.at[pl.ds((w + m * _NW) * _K, _K)],
                             dsts[m], semis[m])

        def zrow(i, carry):
            for k8 in range(8):
                bufs[0][i, pl.ds(k8 * 16, 16)] = jnp.zeros((16,), jnp.float32)
            return carry

        lax.fori_loop(0, _K, zrow, 0)

        @pl.when(s < 14)
        def _():
            for k in range(4):
                pltpu.async_copy(bufs[0], acc_sh.at[pl.ds(s * 624 + k * _K, _K)],
                                 semss[0])
            pltpu.async_copy(bufs[0].at[pl.ds(0, 112)],
                             acc_sh.at[pl.ds(s * 624 + 4 * _K, 112)], semss[0])
            for k in range(4):
                pltpu.make_async_copy(bufs[0], acc_sh.at[pl.ds(s * 624 + k * _K, _K)],
                                      semss[0]).wait()
            pltpu.make_async_copy(bufs[0].at[pl.ds(0, 112)],
                                  acc_sh.at[pl.ds(s * 624 + 4 * _K, 112)],
                                  semss[0]).wait()

        @pl.when(s >= 14)
        def _():
            base = 8736 + (s - 14) * 632
            for k in range(4):
                pltpu.async_copy(bufs[0], acc_sh.at[pl.ds(base + k * _K, _K)],
                                 semss[0])
            pltpu.async_copy(bufs[0].at[pl.ds(0, 120)],
                             acc_sh.at[pl.ds(base + 4 * _K, 120)], semss[0])
            for k in range(4):
                pltpu.make_async_copy(bufs[0], acc_sh.at[pl.ds(base + k * _K, _K)],
                                      semss[0]).wait()
            pltpu.make_async_copy(bufs[0].at[pl.ds(0, 120)],
                                  acc_sh.at[pl.ds(base + 4 * _K, 120)],
                                  semss[0]).wait()

        plsc.subcore_barrier()


        def step(t, carry):
            for m in range(6):
                s3 = m % 3
                p3 = (m - 1) % 3
                s6 = m
                p6 = (m - 1) % 6
                f6 = (m + 3) % 6
                q = 6 * t + m

                def guard(dq, q=q):
                    cond = (w + _NW * (q + dq)) < _EC
                    if m + dq < 0:
                        cond = jnp.logical_and(t > 0, cond)
                    return cond

                @pl.when(guard(-3))
                def _(s3=s3, f6=f6, q=q):
                    pltpu.make_async_copy(
                        bufs[s3], acc_sh.at[dsts[(f6)]], semss[s3]).wait()

                @pl.when(guard(3))
                def _(f6=f6, q=q):
                    nk = w + _NW * (q + 3)
                    pltpu.async_copy(gidx_hbm.at[pl.ds(nk * _K, _K)],
                                     idxs[f6], semis[f6])
                    pltpu.async_copy(dst_hbm.at[pl.ds(nk * _K, _K)],
                                     dsts[f6], semis[f6])

                @pl.when(guard(0))
                def _(s3=s3, s6=s6, q=q):
                    ck = w + _NW * q
                    pltpu.make_async_copy(gidx_hbm.at[pl.ds(ck * _K, _K)],
                                          idxs[s6], semis[s6]).wait()
                    pltpu.make_async_copy(dst_hbm.at[pl.ds(ck * _K, _K)],
                                          dsts[s6], semis[s6]).wait()
                    pltpu.async_copy(table_hbm.at[idxs[s6]], bufs[s3], semgs[s3])

                @pl.when(guard(-1))
                def _(p3=p3, p6=p6):
                    pltpu.make_async_copy(table_hbm.at[idxs[p6]],
                                          bufs[p3], semgs[p3]).wait()
                    pltpu.async_copy(bufs[p3], acc_sh.at[dsts[p6]],
                                     semss[p3], add=True)

            return carry

        lax.fori_loop(0, 14, step, 0)
        plsc.subcore_barrier()

        @pl.when(s < 14)
        def _():
            pltpu.sync_copy(acc_sh.at[pl.ds(s * 624, 624)],
                            out_hbm.at[c, pl.ds(s * 624, 624)])

        @pl.when(s >= 14)
        def _():
            pltpu.sync_copy(acc_sh.at[pl.ds(8736 + (s - 14) * 632, 632)],
                            out_hbm.at[c, pl.ds(8736 + (s - 14) * 632, 632)])

    return body(table, gidx, dst)


def kernel(in_feat, edge_index, edge_types, W1, loop1, b1, W2, loop2, b2):
    gidx, dst = _prep(edge_index.astype(jnp.int32), edge_types.astype(jnp.int32))
    wcat1 = jnp.concatenate([W1, loop1[None]], axis=0)
    wcat2 = jnp.concatenate([W2, loop2[None]], axis=0)
    b1r = b1.reshape(1, _D)
    b2r = b2.reshape(1, _D)

    t1 = _table(in_feat, wcat1)
    p1 = _sc_scatter(t1, gidx, dst)
    t2 = _combine_table(p1, t1, b1r, wcat2)
    p2 = _sc_scatter(t2, gidx, dst)
    return _final(p2, t2, b2r)
